# Initial kernel scaffold; baseline (speedup 1.0000x reference)
import jax
import jax.numpy as jnp
from jax.experimental import pallas as pl


def kernel(query_repr, x, W1, b1, W2, b2, We, be):
    raise NotImplementedError("write your pallas kernel here")


# speedup vs baseline: 5.1809x; 5.1809x over previous
import functools
import jax
import jax.numpy as jnp
from jax.experimental import pallas as pl

B = 2048
H = 1024
HH = 512
E = 16
OUT = 16
EO = E * OUT

BB = 512


def _moe_kernel(q_ref, x_ref, W1_ref, b1_ref, W2_ref, b2_ref,
                WeT_ref, beR_ref, R_ref, S_ref, out_ref):
    q = q_ref[...]
    x = x_ref[...]

    h = jnp.maximum(jnp.dot(q, W1_ref[...], preferred_element_type=jnp.float32)
                    + b1_ref[...], 0.0)
    logits = jnp.dot(h, W2_ref[...], preferred_element_type=jnp.float32) + b2_ref[...]

    col = jax.lax.broadcasted_iota(jnp.int32, logits.shape, 1)
    m0 = jnp.max(logits, axis=1, keepdims=True)
    idx0 = jnp.min(jnp.where(logits == m0, col, E), axis=1, keepdims=True)
    sel0 = col == idx0
    masked = jnp.where(sel0, -jnp.inf, logits)
    m1 = jnp.max(masked, axis=1, keepdims=True)
    idx1 = jnp.min(jnp.where(masked == m1, col, E), axis=1, keepdims=True)
    sel1 = col == idx1
    g0 = jax.nn.sigmoid(m0 - m1)
    gates = jnp.where(sel0, g0, 0.0) + jnp.where(sel1, 1.0 - g0, 0.0)

    y = jnp.dot(x, WeT_ref[...], preferred_element_type=jnp.float32) + beR_ref[...]

    gbig = jnp.dot(gates, R_ref[...], preferred_element_type=jnp.float32)
    out_ref[...] = jnp.dot(gbig * y, S_ref[...], preferred_element_type=jnp.float32)


def kernel(query_repr, x, W1, b1, W2, b2, We, be):
    WeT = We.reshape(EO, H).T
    beR = be.reshape(1, EO)
    e_of = jnp.arange(EO, dtype=jnp.int32) // OUT
    o_of = jnp.arange(EO, dtype=jnp.int32) % OUT
    R = (jnp.arange(E, dtype=jnp.int32)[:, None] == e_of[None, :]).astype(jnp.float32)
    S = (o_of[:, None] == jnp.arange(OUT, dtype=jnp.int32)[None, :]).astype(jnp.float32)

    grid = (B // BB,)
    return pl.pallas_call(
        _moe_kernel,
        grid=grid,
        in_specs=[
            pl.BlockSpec((BB, H), lambda i: (i, 0)),
            pl.BlockSpec((BB, H), lambda i: (i, 0)),
            pl.BlockSpec((H, HH), lambda i: (0, 0)),
            pl.BlockSpec((HH,), lambda i: (0,)),
            pl.BlockSpec((HH, E), lambda i: (0, 0)),
            pl.BlockSpec((E,), lambda i: (0,)),
            pl.BlockSpec((H, EO), lambda i: (0, 0)),
            pl.BlockSpec((1, EO), lambda i: (0, 0)),
            pl.BlockSpec((E, EO), lambda i: (0, 0)),
            pl.BlockSpec((EO, OUT), lambda i: (0, 0)),
        ],
        out_specs=pl.BlockSpec((BB, OUT), lambda i: (i, 0)),
        out_shape=jax.ShapeDtypeStruct((B, OUT), jnp.float32),
    )(query_repr, x, W1, b1, W2, b2, WeT, beR, R, S)
